# baseline (speedup 1.0000x reference)
import functools

import jax
import jax.numpy as jnp
import numpy as np
from jax import lax
from jax.experimental import pallas as pl
from jax.experimental.pallas import tpu as pltpu
from jax.experimental.pallas import tpu_sc as plsc

ROWS = 32
COLS = 8192
K = 4096
LANES = 16
CHUNKS = COLS // LANES
NUM_CORES = 2
UNROLL = 4

_SIGN = np.uint32(0x80000000)


def _hsum(v):
    iota = lax.iota(jnp.int32, LANES)
    for sh in (1, 2, 4, 8):
        v = v + jnp.take(v, (iota + sh) % LANES)
    return v[0]


def _cumsum16(v, iota):
    for sh in (1, 2, 4, 8):
        shifted = jnp.take(v, jnp.maximum(iota - sh, 0))
        v = v + jnp.where(iota >= sh, shifted, 0)
    return v


def _tec_body(z_hbm, g_hbm, out_hbm, zrow, grow, keyrow, outrow):
    wid = lax.axis_index("s") * NUM_CORES + lax.axis_index("c")

    pltpu.sync_copy(z_hbm.at[wid], zrow)
    pltpu.sync_copy(g_hbm.at[wid], grow)

    def radix_pass(prefix, upper, shift, nbits, gen_keys=False):
        ncand = (1 << nbits) - 1
        cands = [prefix | (np.uint32(j) << shift) for j in range(1, ncand + 1)]

        def cbody(i, accs):
            accs = list(accs)
            for u_ in range(UNROLL):
                sl = pl.ds((i * UNROLL + u_) * LANES, LANES)
                if gen_keys:
                    x = zrow[sl] + grow[sl]
                    b = lax.bitcast_convert_type(x, jnp.uint32)
                    kc = jnp.where(b >= _SIGN, ~b, b | _SIGN)
                    keyrow[sl] = kc
                else:
                    kc = keyrow[sl]
                for j in range(ncand):
                    accs[j] = accs[j] + jnp.where(kc >= cands[j], 1, 0).astype(
                        jnp.int32
                    )
            return tuple(accs)

        accs = lax.fori_loop(
            0,
            CHUNKS // UNROLL,
            cbody,
            tuple(jnp.zeros((LANES,), jnp.int32) for _ in range(ncand)),
        )
        cnts = [_hsum(a) for a in accs]
        newprefix = prefix
        for j in range(1, ncand + 1):
            newprefix = jnp.where(cnts[j - 1] >= K, cands[j - 1], newprefix)
        below = [jnp.where(c < K, c, -1) for c in cnts]
        mx = below[0]
        for b in below[1:]:
            mx = jnp.maximum(mx, b)
        upper = jnp.where(mx >= 0, mx, upper)
        return newprefix, upper

    prefix = jnp.uint32(0)
    upper = jnp.int32(0)
    shifts = list(range(29, -1, -3)) + [0]
    for si, shift in enumerate(shifts):
        prefix, upper = radix_pass(
            prefix, upper, np.uint32(shift), 3 if shift else 2, gen_keys=(si == 0)
        )
    t = prefix
    need = K - upper

    def obody(i, acc):
        for u_ in range(UNROLL):
            sl = pl.ds((i * UNROLL + u_) * LANES, LANES)
            kc = keyrow[sl]
            outrow[sl] = jnp.where(kc >= t, jnp.float32(1.0), jnp.float32(0.0))
            acc = acc + jnp.where(kc == t, 1, 0).astype(jnp.int32)
        return acc

    eq_acc = lax.fori_loop(
        0, CHUNKS // UNROLL, obody, jnp.zeros((LANES,), jnp.int32)
    )
    n_eq = _hsum(eq_acc)

    @pl.when(n_eq != need)
    def _tie_fixup():
        iota = lax.iota(jnp.int32, LANES)

        def fbody(i, carry):
            sl = pl.ds(i * LANES, LANES)
            kc = keyrow[sl]
            eqi = jnp.where(kc == t, 1, 0).astype(jnp.int32)
            incl = _cumsum16(eqi, iota)
            keep = jnp.logical_or(kc > t, (carry + incl) <= need)
            outrow[sl] = jnp.where(keep & (kc >= t), jnp.float32(1.0), jnp.float32(0.0))
            return carry + incl[LANES - 1]

        lax.fori_loop(0, CHUNKS, fbody, jnp.int32(0))

    pltpu.sync_copy(outrow, out_hbm.at[wid])


_sc_mask = functools.partial(
    pl.kernel,
    out_type=jax.ShapeDtypeStruct((ROWS, COLS), jnp.float32),
    mesh=plsc.VectorSubcoreMesh(core_axis_name="c", subcore_axis_name="s"),
    scratch_types=[
        pltpu.VMEM((COLS,), jnp.float32),
        pltpu.VMEM((COLS,), jnp.float32),
        pltpu.VMEM((COLS,), jnp.uint32),
        pltpu.VMEM((COLS,), jnp.float32),
    ],
)(_tec_body)


def kernel(z_loga, step):
    del step
    eps = jax.random.uniform(
        jax.random.key(42), z_loga.shape, z_loga.dtype, minval=1e-06, maxval=1 - 1e-06
    )
    gumbel = -jnp.log(-jnp.log(eps))
    return _sc_mask(z_loga, gumbel)


# speedup vs baseline: 1.2928x; 1.2928x over previous
import functools

import jax
import jax.numpy as jnp
import numpy as np
from jax import lax
from jax.experimental import pallas as pl
from jax.experimental.pallas import tpu as pltpu
from jax.experimental.pallas import tpu_sc as plsc

ROWS = 32
COLS = 8192
K = 4096
LANES = 16
CHUNKS = COLS // LANES
NUM_CORES = 2
UNROLL = 1

_SIGN = np.uint32(0x80000000)


def _hsum(v):
    iota = lax.iota(jnp.int32, LANES)
    for sh in (1, 2, 4, 8):
        v = v + jnp.take(v, (iota + sh) % LANES)
    return v[0]


def _cumsum16(v, iota):
    for sh in (1, 2, 4, 8):
        shifted = jnp.take(v, jnp.maximum(iota - sh, 0))
        v = v + jnp.where(iota >= sh, shifted, 0)
    return v


def _tec_body(z_hbm, g_hbm, out_hbm, zrow, grow, keyrow, outrow):
    wid = lax.axis_index("s") * NUM_CORES + lax.axis_index("c")

    pltpu.sync_copy(z_hbm.at[wid], zrow)
    pltpu.sync_copy(g_hbm.at[wid], grow)

    def radix_pass(prefix, upper, shift, nbits, gen_keys=False):
        ncand = (1 << nbits) - 1
        cands = [prefix | (np.uint32(j) << shift) for j in range(1, ncand + 1)]

        def cbody(i, accs):
            accs = list(accs)
            for u_ in range(UNROLL):
                sl = pl.ds((i * UNROLL + u_) * LANES, LANES)
                if gen_keys:
                    x = zrow[sl] + grow[sl]
                    b = lax.bitcast_convert_type(x, jnp.uint32)
                    kc = jnp.where(b >= _SIGN, ~b, b | _SIGN)
                    keyrow[sl] = kc
                else:
                    kc = keyrow[sl]
                for j in range(ncand):
                    accs[j] = accs[j] + jnp.where(kc >= cands[j], 1, 0).astype(
                        jnp.int32
                    )
            return tuple(accs)

        accs = lax.fori_loop(
            0,
            CHUNKS // UNROLL,
            cbody,
            tuple(jnp.zeros((LANES,), jnp.int32) for _ in range(ncand)),
        )
        cnts = [_hsum(a) for a in accs]
        newprefix = prefix
        for j in range(1, ncand + 1):
            newprefix = jnp.where(cnts[j - 1] >= K, cands[j - 1], newprefix)
        below = [jnp.where(c < K, c, -1) for c in cnts]
        mx = below[0]
        for b in below[1:]:
            mx = jnp.maximum(mx, b)
        upper = jnp.where(mx >= 0, mx, upper)
        return newprefix, upper

    prefix = jnp.uint32(0)
    upper = jnp.int32(0)
    shifts = list(range(29, -1, -3)) + [0]
    for si, shift in enumerate(shifts):
        prefix, upper = radix_pass(
            prefix, upper, np.uint32(shift), 3 if shift else 2, gen_keys=(si == 0)
        )
    t = prefix
    need = K - upper

    def obody(i, acc):
        for u_ in range(UNROLL):
            sl = pl.ds((i * UNROLL + u_) * LANES, LANES)
            kc = keyrow[sl]
            outrow[sl] = jnp.where(kc >= t, jnp.float32(1.0), jnp.float32(0.0))
            acc = acc + jnp.where(kc == t, 1, 0).astype(jnp.int32)
        return acc

    eq_acc = lax.fori_loop(
        0, CHUNKS // UNROLL, obody, jnp.zeros((LANES,), jnp.int32)
    )
    n_eq = _hsum(eq_acc)

    @pl.when(n_eq != need)
    def _tie_fixup():
        iota = lax.iota(jnp.int32, LANES)

        def fbody(i, carry):
            sl = pl.ds(i * LANES, LANES)
            kc = keyrow[sl]
            eqi = jnp.where(kc == t, 1, 0).astype(jnp.int32)
            incl = _cumsum16(eqi, iota)
            keep = jnp.logical_or(kc > t, (carry + incl) <= need)
            outrow[sl] = jnp.where(keep & (kc >= t), jnp.float32(1.0), jnp.float32(0.0))
            return carry + incl[LANES - 1]

        lax.fori_loop(0, CHUNKS, fbody, jnp.int32(0))

    pltpu.sync_copy(outrow, out_hbm.at[wid])


_sc_mask = functools.partial(
    pl.kernel,
    out_type=jax.ShapeDtypeStruct((ROWS, COLS), jnp.float32),
    mesh=plsc.VectorSubcoreMesh(core_axis_name="c", subcore_axis_name="s"),
    scratch_types=[
        pltpu.VMEM((COLS,), jnp.float32),
        pltpu.VMEM((COLS,), jnp.float32),
        pltpu.VMEM((COLS,), jnp.uint32),
        pltpu.VMEM((COLS,), jnp.float32),
    ],
)(_tec_body)


def kernel(z_loga, step):
    del step
    eps = jax.random.uniform(
        jax.random.key(42), z_loga.shape, z_loga.dtype, minval=1e-06, maxval=1 - 1e-06
    )
    gumbel = -jnp.log(-jnp.log(eps))
    return _sc_mask(z_loga, gumbel)


# speedup vs baseline: 1.2959x; 1.0024x over previous
import functools

import jax
import jax.numpy as jnp
import numpy as np
from jax import lax
from jax.experimental import pallas as pl
from jax.experimental.pallas import tpu as pltpu
from jax.experimental.pallas import tpu_sc as plsc

ROWS = 32
COLS = 8192
K = 4096
LANES = 16
CHUNKS = COLS // LANES
NUM_CORES = 2
UNROLL = 4

_SIGN = np.uint32(0x80000000)


def _hsum(v):
    iota = lax.iota(jnp.int32, LANES)
    for sh in (1, 2, 4, 8):
        v = v + jnp.take(v, (iota + sh) % LANES)
    return v[0]


def _cumsum16(v, iota):
    for sh in (1, 2, 4, 8):
        shifted = jnp.take(v, jnp.maximum(iota - sh, 0))
        v = v + jnp.where(iota >= sh, shifted, 0)
    return v


def _tec_body(z_hbm, g_hbm, out_hbm, zrow, grow, keyrow, outrow):
    wid = lax.axis_index("s") * NUM_CORES + lax.axis_index("c")

    pltpu.sync_copy(z_hbm.at[wid], zrow)
    pltpu.sync_copy(g_hbm.at[wid], grow)

    def radix_pass(prefix, upper, shift, nbits, gen_keys=False):
        ncand = (1 << nbits) - 1
        cands = [prefix | (np.uint32(j) << shift) for j in range(1, ncand + 1)]

        @plsc.parallel_loop(
            0,
            CHUNKS,
            step=1,
            unroll=UNROLL,
            carry=tuple(jnp.zeros((LANES,), jnp.int32) for _ in range(ncand)),
        )
        def accs(i, accs):
            accs = list(accs)
            sl = pl.ds(i * LANES, LANES)
            if gen_keys:
                x = zrow[sl] + grow[sl]
                b = lax.bitcast_convert_type(x, jnp.uint32)
                kc = jnp.where(b >= _SIGN, ~b, b | _SIGN)
                keyrow[sl] = kc
            else:
                kc = keyrow[sl]
            for j in range(ncand):
                accs[j] = accs[j] + jnp.where(kc >= cands[j], 1, 0).astype(jnp.int32)
            return tuple(accs)
        cnts = [_hsum(a) for a in accs]
        newprefix = prefix
        for j in range(1, ncand + 1):
            newprefix = jnp.where(cnts[j - 1] >= K, cands[j - 1], newprefix)
        below = [jnp.where(c < K, c, -1) for c in cnts]
        mx = below[0]
        for b in below[1:]:
            mx = jnp.maximum(mx, b)
        upper = jnp.where(mx >= 0, mx, upper)
        return newprefix, upper

    prefix = jnp.uint32(0)
    upper = jnp.int32(0)
    shifts = list(range(29, -1, -3)) + [0]
    for si, shift in enumerate(shifts):
        prefix, upper = radix_pass(
            prefix, upper, np.uint32(shift), 3 if shift else 2, gen_keys=(si == 0)
        )
    t = prefix
    need = K - upper

    @plsc.parallel_loop(
        0, CHUNKS, step=1, unroll=UNROLL, carry=jnp.zeros((LANES,), jnp.int32)
    )
    def eq_acc(i, acc):
        sl = pl.ds(i * LANES, LANES)
        kc = keyrow[sl]
        outrow[sl] = jnp.where(kc >= t, jnp.float32(1.0), jnp.float32(0.0))
        return acc + jnp.where(kc == t, 1, 0).astype(jnp.int32)

    n_eq = _hsum(eq_acc)

    @pl.when(n_eq != need)
    def _tie_fixup():
        iota = lax.iota(jnp.int32, LANES)

        def fbody(i, carry):
            sl = pl.ds(i * LANES, LANES)
            kc = keyrow[sl]
            eqi = jnp.where(kc == t, 1, 0).astype(jnp.int32)
            incl = _cumsum16(eqi, iota)
            keep = jnp.logical_or(kc > t, (carry + incl) <= need)
            outrow[sl] = jnp.where(keep & (kc >= t), jnp.float32(1.0), jnp.float32(0.0))
            return carry + incl[LANES - 1]

        lax.fori_loop(0, CHUNKS, fbody, jnp.int32(0))

    pltpu.sync_copy(outrow, out_hbm.at[wid])


_sc_mask = functools.partial(
    pl.kernel,
    out_type=jax.ShapeDtypeStruct((ROWS, COLS), jnp.float32),
    mesh=plsc.VectorSubcoreMesh(core_axis_name="c", subcore_axis_name="s"),
    scratch_types=[
        pltpu.VMEM((COLS,), jnp.float32),
        pltpu.VMEM((COLS,), jnp.float32),
        pltpu.VMEM((COLS,), jnp.uint32),
        pltpu.VMEM((COLS,), jnp.float32),
    ],
)(_tec_body)


def kernel(z_loga, step):
    del step
    eps = jax.random.uniform(
        jax.random.key(42), z_loga.shape, z_loga.dtype, minval=1e-06, maxval=1 - 1e-06
    )
    gumbel = -jnp.log(-jnp.log(eps))
    return _sc_mask(z_loga, gumbel)


# speedup vs baseline: 1.5859x; 1.2238x over previous
import functools

import jax
import jax.numpy as jnp
import numpy as np
from jax import lax
from jax.experimental import pallas as pl
from jax.experimental.pallas import tpu as pltpu
from jax.experimental.pallas import tpu_sc as plsc

ROWS = 32
COLS = 8192
K = 4096
LANES = 16
CHUNKS = COLS // LANES
NUM_CORES = 2
UNROLL = 4

DELTA = np.float32(0.1242)
LADDER = (np.float32(-0.04), np.float32(0.0), np.float32(0.04))
EGT = 2
MAX_INTERP = 12

_SIGN = np.uint32(0x80000000)
_UMAX = np.uint32(0xFFFFFFFF)


def _key_of(v):
    u = lax.bitcast_convert_type(v, jnp.uint32)
    return jnp.where(u >= _SIGN, ~u, u | _SIGN)


def _val_of(k):
    u = jnp.where(k >= _SIGN, k & jnp.uint32(0x7FFFFFFF), ~k)
    return lax.bitcast_convert_type(u, jnp.float32)


def _recip(b):
    t = lax.bitcast_convert_type(
        jnp.int32(0x7EF477D5) - lax.bitcast_convert_type(b, jnp.int32),
        jnp.float32,
    )
    t = t * (jnp.float32(2.0) - b * t)
    t = t * (jnp.float32(2.0) - b * t)
    return t


def _rot_reduce(v, op):
    iota = lax.iota(jnp.int32, LANES)
    for sh in (1, 2, 4, 8):
        v = op(v, jnp.take(v, (iota + sh) % LANES))
    return v


def _splat(x, dtype):
    return jnp.full((LANES,), x, dtype)


def _cumsum16(v, iota):
    for sh in (1, 2, 4, 8):
        shifted = jnp.take(v, jnp.maximum(iota - sh, 0))
        v = v + jnp.where(iota >= sh, shifted, 0)
    return v


def _tec_body(z_hbm, g_hbm, out_hbm, zrow, grow, keyrow, outrow):
    wid = lax.axis_index("s") * NUM_CORES + lax.axis_index("c")

    pltpu.sync_copy(z_hbm.at[wid], zrow)
    pltpu.sync_copy(g_hbm.at[wid], grow)

    @plsc.parallel_loop(
        0,
        CHUNKS,
        step=1,
        unroll=UNROLL,
        carry=(
            jnp.zeros((LANES,), jnp.float32),
            jnp.full((LANES,), _UMAX, jnp.uint32),
            jnp.zeros((LANES,), jnp.uint32),
        ),
    )
    def key_stats(i, carry):
        acc, mn, mx = carry
        sl = pl.ds(i * LANES, LANES)
        x = zrow[sl] + grow[sl]
        kc = _key_of(x)
        keyrow[sl] = kc
        return acc + x, jnp.minimum(mn, kc), jnp.maximum(mx, kc)

    x_acc, key_mn, key_mx = key_stats
    seed_s = _rot_reduce(x_acc, jnp.add)[0] * jnp.float32(1.0 / COLS) - DELTA
    min_key = _rot_reduce(key_mn, jnp.minimum)[0]
    max_key = _rot_reduce(key_mx, jnp.maximum)[0]

    lo = min_key
    clo = jnp.int32(COLS)
    hi = max_key + jnp.uint32(1)
    chi = jnp.int32(0)

    def clamp(cand):
        return jnp.minimum(jnp.maximum(cand, lo + jnp.uint32(1)), hi - jnp.uint32(1))

    lcands = [clamp(_key_of(seed_s + off)) for off in LADDER]

    @plsc.parallel_loop(
        0,
        CHUNKS,
        step=1,
        unroll=UNROLL,
        carry=tuple(jnp.zeros((LANES,), jnp.int32) for _ in LADDER),
    )
    def laccs(i, accs):
        kc = keyrow[pl.ds(i * LANES, LANES)]
        return tuple(
            a + jnp.where(kc >= c_, 1, 0).astype(jnp.int32)
            for a, c_ in zip(accs, lcands)
        )

    for c_, a_ in zip(lcands, laccs):
        n_ = _rot_reduce(a_, jnp.add)[0]
        pl_ = jnp.logical_and(n_ >= K, c_ > lo)
        lo = jnp.where(pl_, c_, lo)
        clo = jnp.where(pl_, n_, clo)
        ph_ = jnp.logical_and(n_ < K, c_ < hi)
        hi = jnp.where(ph_, c_, hi)
        chi = jnp.where(ph_, n_, chi)

    def count_ge(cand):
        def cbody(i, a):
            kc = keyrow[pl.ds(i * LANES, LANES)]
            return a + jnp.where(kc >= cand, 1, 0).astype(jnp.int32)

        acc = lax.fori_loop(0, CHUNKS, cbody, jnp.zeros((LANES,), jnp.int32))
        return _rot_reduce(acc, jnp.add)[0]

    def _pow2_neg(n):
        bits = (jnp.int32(127) - jnp.minimum(n, jnp.int32(60))) << 23
        return lax.bitcast_convert_type(bits, jnp.float32)

    def w_body(i, st):
        def do_probe(st):
            lo, clo, hi, chi, nlo, nhi, last = st
            vlo = _val_of(lo)
            vhi = _val_of(hi - jnp.uint32(1))
            f_lo = (clo - K).astype(jnp.float32) * _pow2_neg(nlo)
            f_hi = (chi - K).astype(jnp.float32) * _pow2_neg(nhi)
            denom = jnp.maximum(f_lo - f_hi, jnp.float32(1e-30))
            frac = jnp.minimum(
                jnp.maximum(f_lo * _recip(denom), jnp.float32(0.0)),
                jnp.float32(1.0),
            )
            cand_i = _key_of(vlo + (vhi - vlo) * frac)
            cand_b = lo + ((hi - lo) >> jnp.uint32(1))
            cand = jnp.minimum(
                jnp.maximum(
                    jnp.where(i < MAX_INTERP, cand_i, cand_b), lo + jnp.uint32(1)
                ),
                hi - jnp.uint32(1),
            )
            c = count_ge(cand)
            pred = c >= K
            nhi = jnp.where(
                pred, jnp.where(last == 1, nhi + 1, jnp.int32(0)), nhi
            )
            nlo = jnp.where(
                pred, nlo, jnp.where(last == -1, nlo + 1, jnp.int32(0))
            )
            lo = jnp.where(pred, cand, lo)
            clo = jnp.where(pred, c, clo)
            hi = jnp.where(pred, hi, cand)
            chi = jnp.where(pred, chi, c)
            last = jnp.where(pred, jnp.int32(1), jnp.int32(-1))
            return (lo, clo, hi, chi, nlo, nhi, last)

        lo, clo, hi, chi = st[0], st[1], st[2], st[3]
        done = jnp.logical_or(clo - K <= EGT, hi - lo <= jnp.uint32(1))
        return lax.cond(done, lambda s: s, do_probe, st)

    lo, clo, hi, chi, _, _, _ = lax.fori_loop(
        0,
        MAX_INTERP + 33,
        w_body,
        (lo, clo, hi, chi, jnp.int32(0), jnp.int32(0), jnp.int32(0)),
    )

    def a_body(_, st):
        def do_step(st):
            P, cP, stop, need = st

            def mbody(i, carry):
                mn, eqc = carry
                kc = keyrow[pl.ds(i * LANES, LANES)]
                mn = jnp.minimum(mn, jnp.where(kc > P, kc, _UMAX))
                eqc = eqc + jnp.where(kc == P, 1, 0).astype(jnp.int32)
                return mn, eqc

            mn, eqc = lax.fori_loop(
                0,
                CHUNKS,
                mbody,
                (
                    jnp.full((LANES,), _UMAX, jnp.uint32),
                    jnp.zeros((LANES,), jnp.int32),
                ),
            )
            M = _rot_reduce(mn, jnp.minimum)[0]
            nP = _rot_reduce(eqc, jnp.add)[0]
            cM = cP - nP
            tie = cM < K
            P = jnp.where(tie, P, M)
            cP = jnp.where(tie, cP, cM)
            need = jnp.where(tie, K - cM, need)
            return (P, cP, tie, need)

        P, cP, stop, need = st
        done = jnp.logical_or(cP <= K, stop)
        return lax.cond(done, lambda s: s, do_step, st)

    t, cP, stop, tie_need = lax.fori_loop(
        0, EGT + 1, a_body, (lo, clo, jnp.bool_(False), jnp.int32(0))
    )

    @plsc.parallel_loop(
        0, CHUNKS, step=1, unroll=UNROLL, carry=jnp.zeros((LANES,), jnp.int32)
    )
    def eq_acc(i, acc):
        sl = pl.ds(i * LANES, LANES)
        kc = keyrow[sl]
        outrow[sl] = jnp.where(kc >= t, jnp.float32(1.0), jnp.float32(0.0))
        return acc + jnp.where(kc == t, 1, 0).astype(jnp.int32)

    n_eq = _rot_reduce(eq_acc, jnp.add)[0]
    need = jnp.where(stop, tie_need, n_eq)

    @pl.when(n_eq != need)
    def _tie_fixup():
        iota = lax.iota(jnp.int32, LANES)

        def fbody(i, carry):
            sl = pl.ds(i * LANES, LANES)
            kc = keyrow[sl]
            eqi = jnp.where(kc == t, 1, 0).astype(jnp.int32)
            incl = _cumsum16(eqi, iota)
            keep = jnp.logical_or(kc > t, (carry + incl) <= need)
            outrow[sl] = jnp.where(
                keep & (kc >= t), jnp.float32(1.0), jnp.float32(0.0)
            )
            return carry + incl[LANES - 1]

        lax.fori_loop(0, CHUNKS, fbody, jnp.int32(0))

    pltpu.sync_copy(outrow, out_hbm.at[wid])


_sc_mask = functools.partial(
    pl.kernel,
    out_type=jax.ShapeDtypeStruct((ROWS, COLS), jnp.float32),
    mesh=plsc.VectorSubcoreMesh(core_axis_name="c", subcore_axis_name="s"),
    scratch_types=[
        pltpu.VMEM((COLS,), jnp.float32),
        pltpu.VMEM((COLS,), jnp.float32),
        pltpu.VMEM((COLS,), jnp.uint32),
        pltpu.VMEM((COLS,), jnp.float32),
    ],
)(_tec_body)


def kernel(z_loga, step):
    del step
    eps = jax.random.uniform(
        jax.random.key(42), z_loga.shape, z_loga.dtype, minval=1e-06, maxval=1 - 1e-06
    )
    gumbel = -jnp.log(-jnp.log(eps))
    return _sc_mask(z_loga, gumbel)


# speedup vs baseline: 1.9268x; 1.2150x over previous
import functools

import jax
import jax.numpy as jnp
import numpy as np
from jax import lax
from jax.experimental import pallas as pl
from jax.experimental.pallas import tpu as pltpu
from jax.experimental.pallas import tpu_sc as plsc

ROWS = 32
COLS = 8192
K = 4096
LANES = 16
CHUNKS = COLS // LANES
NUM_CORES = 2
UNROLL = 4

DELTA = np.float32(0.1242)
LADDER = (np.float32(-0.04), np.float32(0.0), np.float32(0.04))
EGT = 2
MAX_INTERP = 12

_SIGN = np.uint32(0x80000000)
_UMAX = np.uint32(0xFFFFFFFF)


def _key_of(v):
    u = lax.bitcast_convert_type(v, jnp.uint32)
    return jnp.where(u >= _SIGN, ~u, u | _SIGN)


def _val_of(k):
    u = jnp.where(k >= _SIGN, k & jnp.uint32(0x7FFFFFFF), ~k)
    return lax.bitcast_convert_type(u, jnp.float32)


def _recip(b):
    t = lax.bitcast_convert_type(
        jnp.int32(0x7EF477D5) - lax.bitcast_convert_type(b, jnp.int32),
        jnp.float32,
    )
    t = t * (jnp.float32(2.0) - b * t)
    t = t * (jnp.float32(2.0) - b * t)
    return t


def _rot_reduce(v, op):
    iota = lax.iota(jnp.int32, LANES)
    for sh in (1, 2, 4, 8):
        v = op(v, jnp.take(v, (iota + sh) % LANES))
    return v


def _splat(x, dtype):
    return jnp.full((LANES,), x, dtype)


def _cumsum16(v, iota):
    for sh in (1, 2, 4, 8):
        shifted = jnp.take(v, jnp.maximum(iota - sh, 0))
        v = v + jnp.where(iota >= sh, shifted, 0)
    return v


def _tec_body(z_hbm, g_hbm, out_hbm, zrow, grow, keyrow, outrow):
    wid = lax.axis_index("s") * NUM_CORES + lax.axis_index("c")

    pltpu.sync_copy(z_hbm.at[wid], zrow)
    pltpu.sync_copy(g_hbm.at[wid], grow)

    @plsc.parallel_loop(
        0,
        CHUNKS,
        step=1,
        unroll=UNROLL,
        carry=(
            jnp.zeros((LANES,), jnp.float32),
            jnp.full((LANES,), _UMAX, jnp.uint32),
            jnp.zeros((LANES,), jnp.uint32),
        ),
    )
    def key_stats(i, carry):
        acc, mn, mx = carry
        sl = pl.ds(i * LANES, LANES)
        x = zrow[sl] + grow[sl]
        kc = _key_of(x)
        keyrow[sl] = kc
        return acc + x, jnp.minimum(mn, kc), jnp.maximum(mx, kc)

    x_acc, key_mn, key_mx = key_stats
    seed_s = _rot_reduce(x_acc, jnp.add)[0] * jnp.float32(1.0 / COLS) - DELTA
    min_key = _rot_reduce(key_mn, jnp.minimum)[0]
    max_key = _rot_reduce(key_mx, jnp.maximum)[0]

    lo = min_key
    clo = jnp.int32(COLS)
    hi = max_key + jnp.uint32(1)
    chi = jnp.int32(0)

    def clamp(cand):
        return jnp.minimum(jnp.maximum(cand, lo + jnp.uint32(1)), hi - jnp.uint32(1))

    lcands = [clamp(_key_of(seed_s + off)) for off in LADDER]

    @plsc.parallel_loop(
        0,
        CHUNKS,
        step=1,
        unroll=UNROLL,
        carry=tuple(jnp.zeros((LANES,), jnp.int32) for _ in LADDER),
    )
    def laccs(i, accs):
        kc = keyrow[pl.ds(i * LANES, LANES)]
        return tuple(
            a + jnp.where(kc >= c_, 1, 0).astype(jnp.int32)
            for a, c_ in zip(accs, lcands)
        )

    for c_, a_ in zip(lcands, laccs):
        n_ = _rot_reduce(a_, jnp.add)[0]
        pl_ = jnp.logical_and(n_ >= K, c_ > lo)
        lo = jnp.where(pl_, c_, lo)
        clo = jnp.where(pl_, n_, clo)
        ph_ = jnp.logical_and(n_ < K, c_ < hi)
        hi = jnp.where(ph_, c_, hi)
        chi = jnp.where(ph_, n_, chi)

    def count_ge(cand):
        def cbody(i, a):
            for u_ in range(UNROLL):
                kc = keyrow[pl.ds((i * UNROLL + u_) * LANES, LANES)]
                a = a + jnp.where(kc >= cand, 1, 0).astype(jnp.int32)
            return a

        acc = lax.fori_loop(
            0, CHUNKS // UNROLL, cbody, jnp.zeros((LANES,), jnp.int32)
        )
        return _rot_reduce(acc, jnp.add)[0]

    def _pow2_neg(n):
        bits = (jnp.int32(127) - jnp.minimum(n, jnp.int32(60))) << 23
        return lax.bitcast_convert_type(bits, jnp.float32)

    def w_body(i, st):
        def do_probe(st):
            lo, clo, hi, chi, nlo, nhi, last = st
            vlo = _val_of(lo)
            vhi = _val_of(hi - jnp.uint32(1))
            f_lo = (clo - K).astype(jnp.float32) * _pow2_neg(nlo)
            f_hi = (chi - K).astype(jnp.float32) * _pow2_neg(nhi)
            denom = jnp.maximum(f_lo - f_hi, jnp.float32(1e-30))
            frac = jnp.minimum(
                jnp.maximum(f_lo * _recip(denom), jnp.float32(0.0)),
                jnp.float32(1.0),
            )
            cand_i = _key_of(vlo + (vhi - vlo) * frac)
            cand_b = lo + ((hi - lo) >> jnp.uint32(1))
            cand = jnp.minimum(
                jnp.maximum(
                    jnp.where(i < MAX_INTERP, cand_i, cand_b), lo + jnp.uint32(1)
                ),
                hi - jnp.uint32(1),
            )
            c = count_ge(cand)
            pred = c >= K
            nhi = jnp.where(
                pred, jnp.where(last == 1, nhi + 1, jnp.int32(0)), nhi
            )
            nlo = jnp.where(
                pred, nlo, jnp.where(last == -1, nlo + 1, jnp.int32(0))
            )
            lo = jnp.where(pred, cand, lo)
            clo = jnp.where(pred, c, clo)
            hi = jnp.where(pred, hi, cand)
            chi = jnp.where(pred, chi, c)
            last = jnp.where(pred, jnp.int32(1), jnp.int32(-1))
            return (lo, clo, hi, chi, nlo, nhi, last)

        lo, clo, hi, chi = st[0], st[1], st[2], st[3]
        done = jnp.logical_or(clo - K <= EGT, hi - lo <= jnp.uint32(1))
        return lax.cond(done, lambda s: s, do_probe, st)

    lo, clo, hi, chi, _, _, _ = lax.fori_loop(
        0,
        MAX_INTERP + 33,
        w_body,
        (lo, clo, hi, chi, jnp.int32(0), jnp.int32(0), jnp.int32(0)),
    )

    def a_body(_, st):
        def do_step(st):
            P, cP, stop, need = st

            def mbody(i, carry):
                mn, eqc = carry
                for u_ in range(UNROLL):
                    kc = keyrow[pl.ds((i * UNROLL + u_) * LANES, LANES)]
                    mn = jnp.minimum(mn, jnp.where(kc > P, kc, _UMAX))
                    eqc = eqc + jnp.where(kc == P, 1, 0).astype(jnp.int32)
                return mn, eqc

            mn, eqc = lax.fori_loop(
                0,
                CHUNKS // UNROLL,
                mbody,
                (
                    jnp.full((LANES,), _UMAX, jnp.uint32),
                    jnp.zeros((LANES,), jnp.int32),
                ),
            )
            M = _rot_reduce(mn, jnp.minimum)[0]
            nP = _rot_reduce(eqc, jnp.add)[0]
            cM = cP - nP
            tie = cM < K
            P = jnp.where(tie, P, M)
            cP = jnp.where(tie, cP, cM)
            need = jnp.where(tie, K - cM, need)
            return (P, cP, tie, need)

        P, cP, stop, need = st
        done = jnp.logical_or(cP <= K, stop)
        return lax.cond(done, lambda s: s, do_step, st)

    t, cP, stop, tie_need = lax.fori_loop(
        0, EGT + 1, a_body, (lo, clo, jnp.bool_(False), jnp.int32(0))
    )

    @plsc.parallel_loop(
        0, CHUNKS, step=1, unroll=UNROLL, carry=jnp.zeros((LANES,), jnp.int32)
    )
    def eq_acc(i, acc):
        sl = pl.ds(i * LANES, LANES)
        kc = keyrow[sl]
        outrow[sl] = jnp.where(kc >= t, jnp.float32(1.0), jnp.float32(0.0))
        return acc + jnp.where(kc == t, 1, 0).astype(jnp.int32)

    n_eq = _rot_reduce(eq_acc, jnp.add)[0]
    need = jnp.where(stop, tie_need, n_eq)

    @pl.when(n_eq != need)
    def _tie_fixup():
        iota = lax.iota(jnp.int32, LANES)

        def fbody(i, carry):
            sl = pl.ds(i * LANES, LANES)
            kc = keyrow[sl]
            eqi = jnp.where(kc == t, 1, 0).astype(jnp.int32)
            incl = _cumsum16(eqi, iota)
            keep = jnp.logical_or(kc > t, (carry + incl) <= need)
            outrow[sl] = jnp.where(
                keep & (kc >= t), jnp.float32(1.0), jnp.float32(0.0)
            )
            return carry + incl[LANES - 1]

        lax.fori_loop(0, CHUNKS, fbody, jnp.int32(0))

    pltpu.sync_copy(outrow, out_hbm.at[wid])


_sc_mask = functools.partial(
    pl.kernel,
    out_type=jax.ShapeDtypeStruct((ROWS, COLS), jnp.float32),
    mesh=plsc.VectorSubcoreMesh(core_axis_name="c", subcore_axis_name="s"),
    scratch_types=[
        pltpu.VMEM((COLS,), jnp.float32),
        pltpu.VMEM((COLS,), jnp.float32),
        pltpu.VMEM((COLS,), jnp.uint32),
        pltpu.VMEM((COLS,), jnp.float32),
    ],
)(_tec_body)


def kernel(z_loga, step):
    del step
    eps = jax.random.uniform(
        jax.random.key(42), z_loga.shape, z_loga.dtype, minval=1e-06, maxval=1 - 1e-06
    )
    gumbel = -jnp.log(-jnp.log(eps))
    return _sc_mask(z_loga, gumbel)


# speedup vs baseline: 1.9531x; 1.0136x over previous
import functools

import jax
import jax.numpy as jnp
import numpy as np
from jax import lax
from jax.experimental import pallas as pl
from jax.experimental.pallas import tpu as pltpu
from jax.experimental.pallas import tpu_sc as plsc

ROWS = 32
COLS = 8192
K = 4096
LANES = 16
CHUNKS = COLS // LANES
NUM_CORES = 2
UNROLL = 8

DELTA = np.float32(0.1242)
LADDER = (np.float32(-0.04), np.float32(0.0), np.float32(0.04))
EGT = 2
MAX_INTERP = 12

_SIGN = np.uint32(0x80000000)
_UMAX = np.uint32(0xFFFFFFFF)


def _key_of(v):
    u = lax.bitcast_convert_type(v, jnp.uint32)
    return jnp.where(u >= _SIGN, ~u, u | _SIGN)


def _val_of(k):
    u = jnp.where(k >= _SIGN, k & jnp.uint32(0x7FFFFFFF), ~k)
    return lax.bitcast_convert_type(u, jnp.float32)


def _recip(b):
    t = lax.bitcast_convert_type(
        jnp.int32(0x7EF477D5) - lax.bitcast_convert_type(b, jnp.int32),
        jnp.float32,
    )
    t = t * (jnp.float32(2.0) - b * t)
    t = t * (jnp.float32(2.0) - b * t)
    return t


def _rot_reduce(v, op):
    iota = lax.iota(jnp.int32, LANES)
    for sh in (1, 2, 4, 8):
        v = op(v, jnp.take(v, (iota + sh) % LANES))
    return v


def _splat(x, dtype):
    return jnp.full((LANES,), x, dtype)


def _cumsum16(v, iota):
    for sh in (1, 2, 4, 8):
        shifted = jnp.take(v, jnp.maximum(iota - sh, 0))
        v = v + jnp.where(iota >= sh, shifted, 0)
    return v


def _tec_body(z_hbm, g_hbm, out_hbm, zrow, grow, keyrow, outrow):
    wid = lax.axis_index("s") * NUM_CORES + lax.axis_index("c")

    pltpu.sync_copy(z_hbm.at[wid], zrow)
    pltpu.sync_copy(g_hbm.at[wid], grow)

    @plsc.parallel_loop(
        0,
        CHUNKS,
        step=1,
        unroll=UNROLL,
        carry=(
            jnp.zeros((LANES,), jnp.float32),
            jnp.full((LANES,), _UMAX, jnp.uint32),
            jnp.zeros((LANES,), jnp.uint32),
        ),
    )
    def key_stats(i, carry):
        acc, mn, mx = carry
        sl = pl.ds(i * LANES, LANES)
        x = zrow[sl] + grow[sl]
        kc = _key_of(x)
        keyrow[sl] = kc
        return acc + x, jnp.minimum(mn, kc), jnp.maximum(mx, kc)

    x_acc, key_mn, key_mx = key_stats
    seed_s = _rot_reduce(x_acc, jnp.add)[0] * jnp.float32(1.0 / COLS) - DELTA
    min_key = _rot_reduce(key_mn, jnp.minimum)[0]
    max_key = _rot_reduce(key_mx, jnp.maximum)[0]

    lo = min_key
    clo = jnp.int32(COLS)
    hi = max_key + jnp.uint32(1)
    chi = jnp.int32(0)

    def clamp(cand):
        return jnp.minimum(jnp.maximum(cand, lo + jnp.uint32(1)), hi - jnp.uint32(1))

    lcands = [clamp(_key_of(seed_s + off)) for off in LADDER]

    @plsc.parallel_loop(
        0,
        CHUNKS,
        step=1,
        unroll=UNROLL,
        carry=tuple(jnp.zeros((LANES,), jnp.int32) for _ in LADDER),
    )
    def laccs(i, accs):
        kc = keyrow[pl.ds(i * LANES, LANES)]
        return tuple(
            a + jnp.where(kc >= c_, 1, 0).astype(jnp.int32)
            for a, c_ in zip(accs, lcands)
        )

    for c_, a_ in zip(lcands, laccs):
        n_ = _rot_reduce(a_, jnp.add)[0]
        pl_ = jnp.logical_and(n_ >= K, c_ > lo)
        lo = jnp.where(pl_, c_, lo)
        clo = jnp.where(pl_, n_, clo)
        ph_ = jnp.logical_and(n_ < K, c_ < hi)
        hi = jnp.where(ph_, c_, hi)
        chi = jnp.where(ph_, n_, chi)

    def count_ge(cand):
        def cbody(i, a):
            for u_ in range(UNROLL):
                kc = keyrow[pl.ds((i * UNROLL + u_) * LANES, LANES)]
                a = a + jnp.where(kc >= cand, 1, 0).astype(jnp.int32)
            return a

        acc = lax.fori_loop(
            0, CHUNKS // UNROLL, cbody, jnp.zeros((LANES,), jnp.int32)
        )
        return _rot_reduce(acc, jnp.add)[0]

    def _pow2_neg(n):
        bits = (jnp.int32(127) - jnp.minimum(n, jnp.int32(60))) << 23
        return lax.bitcast_convert_type(bits, jnp.float32)

    def w_body(i, st):
        def do_probe(st):
            lo, clo, hi, chi, nlo, nhi, last = st
            vlo = _val_of(lo)
            vhi = _val_of(hi - jnp.uint32(1))
            f_lo = (clo - K).astype(jnp.float32) * _pow2_neg(nlo)
            f_hi = (chi - K).astype(jnp.float32) * _pow2_neg(nhi)
            denom = jnp.maximum(f_lo - f_hi, jnp.float32(1e-30))
            frac = jnp.minimum(
                jnp.maximum(f_lo * _recip(denom), jnp.float32(0.0)),
                jnp.float32(1.0),
            )
            cand_i = _key_of(vlo + (vhi - vlo) * frac)
            cand_b = lo + ((hi - lo) >> jnp.uint32(1))
            cand = jnp.minimum(
                jnp.maximum(
                    jnp.where(i < MAX_INTERP, cand_i, cand_b), lo + jnp.uint32(1)
                ),
                hi - jnp.uint32(1),
            )
            c = count_ge(cand)
            pred = c >= K
            nhi = jnp.where(
                pred, jnp.where(last == 1, nhi + 1, jnp.int32(0)), nhi
            )
            nlo = jnp.where(
                pred, nlo, jnp.where(last == -1, nlo + 1, jnp.int32(0))
            )
            lo = jnp.where(pred, cand, lo)
            clo = jnp.where(pred, c, clo)
            hi = jnp.where(pred, hi, cand)
            chi = jnp.where(pred, chi, c)
            last = jnp.where(pred, jnp.int32(1), jnp.int32(-1))
            return (lo, clo, hi, chi, nlo, nhi, last)

        lo, clo, hi, chi = st[0], st[1], st[2], st[3]
        done = jnp.logical_or(clo - K <= EGT, hi - lo <= jnp.uint32(1))
        return lax.cond(done, lambda s: s, do_probe, st)

    lo, clo, hi, chi, _, _, _ = lax.fori_loop(
        0,
        MAX_INTERP + 33,
        w_body,
        (lo, clo, hi, chi, jnp.int32(0), jnp.int32(0), jnp.int32(0)),
    )

    def a_body(_, st):
        def do_step(st):
            P, cP, stop, need = st

            def mbody(i, carry):
                mn, eqc = carry
                for u_ in range(UNROLL):
                    kc = keyrow[pl.ds((i * UNROLL + u_) * LANES, LANES)]
                    mn = jnp.minimum(mn, jnp.where(kc > P, kc, _UMAX))
                    eqc = eqc + jnp.where(kc == P, 1, 0).astype(jnp.int32)
                return mn, eqc

            mn, eqc = lax.fori_loop(
                0,
                CHUNKS // UNROLL,
                mbody,
                (
                    jnp.full((LANES,), _UMAX, jnp.uint32),
                    jnp.zeros((LANES,), jnp.int32),
                ),
            )
            M = _rot_reduce(mn, jnp.minimum)[0]
            nP = _rot_reduce(eqc, jnp.add)[0]
            cM = cP - nP
            tie = cM < K
            P = jnp.where(tie, P, M)
            cP = jnp.where(tie, cP, cM)
            need = jnp.where(tie, K - cM, need)
            return (P, cP, tie, need)

        P, cP, stop, need = st
        done = jnp.logical_or(cP <= K, stop)
        return lax.cond(done, lambda s: s, do_step, st)

    t, cP, stop, tie_need = lax.fori_loop(
        0, EGT + 1, a_body, (lo, clo, jnp.bool_(False), jnp.int32(0))
    )

    @plsc.parallel_loop(
        0, CHUNKS, step=1, unroll=UNROLL, carry=jnp.zeros((LANES,), jnp.int32)
    )
    def eq_acc(i, acc):
        sl = pl.ds(i * LANES, LANES)
        kc = keyrow[sl]
        outrow[sl] = jnp.where(kc >= t, jnp.float32(1.0), jnp.float32(0.0))
        return acc + jnp.where(kc == t, 1, 0).astype(jnp.int32)

    n_eq = _rot_reduce(eq_acc, jnp.add)[0]
    need = jnp.where(stop, tie_need, n_eq)

    @pl.when(n_eq != need)
    def _tie_fixup():
        iota = lax.iota(jnp.int32, LANES)

        def fbody(i, carry):
            sl = pl.ds(i * LANES, LANES)
            kc = keyrow[sl]
            eqi = jnp.where(kc == t, 1, 0).astype(jnp.int32)
            incl = _cumsum16(eqi, iota)
            keep = jnp.logical_or(kc > t, (carry + incl) <= need)
            outrow[sl] = jnp.where(
                keep & (kc >= t), jnp.float32(1.0), jnp.float32(0.0)
            )
            return carry + incl[LANES - 1]

        lax.fori_loop(0, CHUNKS, fbody, jnp.int32(0))

    pltpu.sync_copy(outrow, out_hbm.at[wid])


_sc_mask = functools.partial(
    pl.kernel,
    out_type=jax.ShapeDtypeStruct((ROWS, COLS), jnp.float32),
    mesh=plsc.VectorSubcoreMesh(core_axis_name="c", subcore_axis_name="s"),
    scratch_types=[
        pltpu.VMEM((COLS,), jnp.float32),
        pltpu.VMEM((COLS,), jnp.float32),
        pltpu.VMEM((COLS,), jnp.uint32),
        pltpu.VMEM((COLS,), jnp.float32),
    ],
)(_tec_body)


def kernel(z_loga, step):
    del step
    eps = jax.random.uniform(
        jax.random.key(42), z_loga.shape, z_loga.dtype, minval=1e-06, maxval=1 - 1e-06
    )
    gumbel = -jnp.log(-jnp.log(eps))
    return _sc_mask(z_loga, gumbel)
